# baseline (speedup 1.0000x reference)
import functools

import jax
import jax.numpy as jnp
from jax import lax
from jax.experimental import pallas as pl
from jax.experimental.pallas import tpu as pltpu
from jax.experimental.pallas import tpu_sc as plsc

BATCH, SEQ = 4096, 200
HIDDEN = 128
HALF = HIDDEN // 2
N = BATCH * SEQ
NC, NS = 2, 16
NW = NC * NS
PER_W = N // NW
T = 128
CHUNKS = PER_W // T
L = 16
NBUF = 6


def _embed_body(i0_hbm, i1_hbm, i2_hbm, x_hbm, y_hbm, tlo_hbm, thi_hbm,
                out_hbm, idx0, idx1, idx2, lobuf, hibuf,
                ssem, tsem, asem, osem):
    wid = lax.axis_index("s") * NC + lax.axis_index("c")
    base = wid * PER_W

    def stage(c):
        p = c % NBUF
        src = pl.ds(base + c * T, T)
        pltpu.async_copy(i0_hbm.at[src], idx0.at[p], ssem.at[p])
        pltpu.async_copy(i1_hbm.at[src], idx1.at[p], ssem.at[p])
        pltpu.async_copy(i2_hbm.at[src], idx2.at[p], ssem.at[p])

    def wait_stage(p):
        dummy = pl.ds(0, T)
        pltpu.make_async_copy(i0_hbm.at[dummy], idx0.at[p], ssem.at[p]).wait()
        pltpu.make_async_copy(i1_hbm.at[dummy], idx1.at[p], ssem.at[p]).wait()
        pltpu.make_async_copy(i2_hbm.at[dummy], idx2.at[p], ssem.at[p]).wait()

    def bump(p):
        def bbody(i, carry):
            s = pl.ds(i * L, L)
            idx0[p, s] = idx0[p, s] + 1
            idx1[p, s] = idx1[p, s] + 1
            idx2[p, s] = idx2[p, s] + 1
            return carry
        lax.fori_loop(0, T // L, bbody, 0, unroll=True)

    def fire_writes(c):
        p = c % NBUF
        pltpu.async_copy(x_hbm.at[idx0.at[p]], lobuf.at[p], tsem.at[p])
        pltpu.async_copy(y_hbm.at[idx1.at[p]], hibuf.at[p], tsem.at[p])

    def wait_writes(p):
        pltpu.make_async_copy(x_hbm.at[idx0.at[p]], lobuf.at[p],
                              tsem.at[p]).wait()
        pltpu.make_async_copy(y_hbm.at[idx1.at[p]], hibuf.at[p],
                              tsem.at[p]).wait()

    def fire_adds(c):
        p = c % NBUF
        pltpu.async_copy(tlo_hbm.at[idx2.at[p]], lobuf.at[p], asem.at[p],
                         add=True)
        pltpu.async_copy(thi_hbm.at[idx2.at[p]], hibuf.at[p], asem.at[p],
                         add=True)

    def wait_adds(p):
        pltpu.make_async_copy(tlo_hbm.at[idx2.at[p]], lobuf.at[p],
                              asem.at[p]).wait()
        pltpu.make_async_copy(thi_hbm.at[idx2.at[p]], hibuf.at[p],
                              asem.at[p]).wait()

    def fire_store(c):
        p = c % NBUF
        rows = pl.ds(base + c * T, T)
        pltpu.async_copy(lobuf.at[p], out_hbm.at[rows, pl.ds(0, HALF)],
                         osem.at[p])
        pltpu.async_copy(hibuf.at[p], out_hbm.at[rows, pl.ds(HALF, HALF)],
                         osem.at[p])

    def wait_store(p):
        rows = pl.ds(base, T)
        pltpu.make_async_copy(lobuf.at[p], out_hbm.at[rows, pl.ds(0, HALF)],
                              osem.at[p]).wait()
        pltpu.make_async_copy(hibuf.at[p], out_hbm.at[rows, pl.ds(HALF, HALF)],
                              osem.at[p]).wait()

    stage(0)
    stage(1)

    def it(c, carry):
        p = c % NBUF

        @pl.when(jnp.logical_and(c >= 1, c <= CHUNKS))
        def _adds():
            q = (c - 1) % NBUF
            wait_writes(q)
            fire_adds(c - 1)

        @pl.when(c >= 2)
        def _store():
            r = (c - 2) % NBUF
            wait_adds(r)
            fire_store(c - 2)

        @pl.when(c < CHUNKS)
        def _front():
            wait_stage(p)
            bump(p)

            @pl.when(c >= NBUF)
            def _reuse():
                wait_store(p)

            fire_writes(c)

            @pl.when(c + 2 < CHUNKS)
            def _stage_ahead():
                stage(c + 2)

        return carry

    lax.fori_loop(0, CHUNKS + 2, it, 0)

    for k in range(CHUNKS - NBUF, CHUNKS):
        wait_store(k % NBUF)


@jax.jit
def _embed(i0, i1, i2, x_embedding, y_embedding, tlo, thi):
    mesh = plsc.VectorSubcoreMesh(core_axis_name="c", subcore_axis_name="s",
                                  num_cores=NC, num_subcores=NS)
    f = pl.kernel(
        _embed_body,
        out_type=jax.ShapeDtypeStruct((N, HIDDEN), jnp.bfloat16),
        mesh=mesh,
        compiler_params=pltpu.CompilerParams(use_tc_tiling_on_sc=False),
        scratch_types=[
            pltpu.VMEM((NBUF, T), jnp.int32),
            pltpu.VMEM((NBUF, T), jnp.int32),
            pltpu.VMEM((NBUF, T), jnp.int32),
            pltpu.VMEM((NBUF, T, HALF), jnp.bfloat16),
            pltpu.VMEM((NBUF, T, HALF), jnp.bfloat16),
            pltpu.SemaphoreType.DMA((NBUF,)),
            pltpu.SemaphoreType.DMA((NBUF,)),
            pltpu.SemaphoreType.DMA((NBUF,)),
            pltpu.SemaphoreType.DMA((NBUF,)),
        ],
    )
    return f(i0, i1, i2, x_embedding, y_embedding, tlo, thi)


CONV_TB = 2048


def _conv_body(in_ref, out_ref):
    out_ref[...] = in_ref[...].astype(jnp.float32)


def _upconvert(out_bf16):
    return pl.pallas_call(
        _conv_body,
        grid=(N // CONV_TB,),
        in_specs=[pl.BlockSpec((CONV_TB, HIDDEN), lambda i: (i, 0))],
        out_specs=pl.BlockSpec((CONV_TB, HIDDEN), lambda i: (i, 0)),
        out_shape=jax.ShapeDtypeStruct((N, HIDDEN), jnp.float32),
    )(out_bf16)


def kernel(input_states, x_embedding, y_embedding, type_embedding):
    inp = input_states.reshape(N, 3).astype(jnp.int32)
    i0 = inp[:, 0]
    i1 = inp[:, 1]
    i2 = inp[:, 2]
    xb = x_embedding.astype(jnp.bfloat16)
    yb = y_embedding.astype(jnp.bfloat16)
    tlo = type_embedding[:, :HALF].astype(jnp.bfloat16)
    thi = type_embedding[:, HALF:].astype(jnp.bfloat16)
    out = _embed(i0, i1, i2, xb, yb, tlo, thi)
    return _upconvert(out).reshape(BATCH, SEQ, HIDDEN)


# speedup vs baseline: 2.0279x; 2.0279x over previous
import functools

import jax
import jax.numpy as jnp
from jax import lax
from jax.experimental import pallas as pl
from jax.experimental.pallas import tpu as pltpu
from jax.experimental.pallas import tpu_sc as plsc

BATCH, SEQ = 4096, 200
HIDDEN = 128
HALF = HIDDEN // 2
N = BATCH * SEQ
NC, NS = 2, 16
NW = NC * NS
PER_W = N // NW
T = 128
CHUNKS = PER_W // T
L = 16
NBUF = 4


def _embed_body(i0_hbm, i1_hbm, i2_hbm, x_hbm, y_hbm, tlo_hbm, thi_hbm,
                out_hbm, idx0, idx1, idx2, lobuf, hibuf, obuf,
                ssem, tsem, asem, osem):
    wid = lax.axis_index("s") * NC + lax.axis_index("c")
    base = wid * PER_W

    def stage(c):
        p = c % NBUF
        src = pl.ds(base + c * T, T)
        pltpu.async_copy(i0_hbm.at[src], idx0.at[p], ssem.at[p])
        pltpu.async_copy(i1_hbm.at[src], idx1.at[p], ssem.at[p])
        pltpu.async_copy(i2_hbm.at[src], idx2.at[p], ssem.at[p])

    def wait_stage(p):
        dummy = pl.ds(0, T)
        pltpu.make_async_copy(i0_hbm.at[dummy], idx0.at[p], ssem.at[p]).wait()
        pltpu.make_async_copy(i1_hbm.at[dummy], idx1.at[p], ssem.at[p]).wait()
        pltpu.make_async_copy(i2_hbm.at[dummy], idx2.at[p], ssem.at[p]).wait()

    def bump(p):
        def bbody(i, carry):
            s = pl.ds(i * L, L)
            idx0[p, s] = idx0[p, s] + 1
            idx1[p, s] = idx1[p, s] + 1
            idx2[p, s] = idx2[p, s] + 1
            return carry
        lax.fori_loop(0, T // L, bbody, 0, unroll=True)

    def fire_writes(c):
        p = c % NBUF
        pltpu.async_copy(x_hbm.at[idx0.at[p]], lobuf.at[p], tsem.at[p])
        pltpu.async_copy(y_hbm.at[idx1.at[p]], hibuf.at[p], tsem.at[p])

    def wait_writes(p):
        pltpu.make_async_copy(x_hbm.at[idx0.at[p]], lobuf.at[p],
                              tsem.at[p]).wait()
        pltpu.make_async_copy(y_hbm.at[idx1.at[p]], hibuf.at[p],
                              tsem.at[p]).wait()

    def fire_adds(c):
        p = c % NBUF
        pltpu.async_copy(tlo_hbm.at[idx2.at[p]], lobuf.at[p], asem.at[p],
                         add=True)
        pltpu.async_copy(thi_hbm.at[idx2.at[p]], hibuf.at[p], asem.at[p],
                         add=True)

    def wait_adds(p):
        pltpu.make_async_copy(tlo_hbm.at[idx2.at[p]], lobuf.at[p],
                              asem.at[p]).wait()
        pltpu.make_async_copy(thi_hbm.at[idx2.at[p]], hibuf.at[p],
                              asem.at[p]).wait()

    def combine(r):
        def vbody(t, carry):
            for j in range(2):
                ab = lobuf[r, t, pl.ds(32 * j, 32)]
                a, b = plsc.unpack(ab, format=plsc.PackFormat.INTERLEAVED)
                obuf[r, t, pl.ds(32 * j, L)] = a
                obuf[r, t, pl.ds(32 * j + L, L)] = b
                cd = hibuf[r, t, pl.ds(32 * j, 32)]
                cc, dd = plsc.unpack(cd, format=plsc.PackFormat.INTERLEAVED)
                obuf[r, t, pl.ds(HALF + 32 * j, L)] = cc
                obuf[r, t, pl.ds(HALF + 32 * j + L, L)] = dd
            return carry

        lax.fori_loop(0, T, vbody, 0, unroll=4)

    def fire_store(c):
        p = c % NBUF
        pltpu.async_copy(obuf.at[p], out_hbm.at[pl.ds(base + c * T, T)],
                         osem.at[p])

    def wait_store(p):
        pltpu.make_async_copy(obuf.at[p], out_hbm.at[pl.ds(base, T)],
                              osem.at[p]).wait()

    stage(0)
    stage(1)

    def it(c, carry):
        @pl.when(jnp.logical_and(c >= 1, c <= CHUNKS))
        def _adds():
            q = (c - 1) % NBUF
            wait_writes(q)
            fire_adds(c - 1)

        @pl.when(c >= 2)
        def _back():
            r = (c - 2) % NBUF
            wait_adds(r)
            combine(r)
            fire_store(c - 2)

        @pl.when(c < CHUNKS)
        def _front():
            p = c % NBUF
            wait_stage(p)
            bump(p)

            @pl.when(c >= NBUF)
            def _reuse():
                wait_store(p)

            fire_writes(c)

            @pl.when(c + 2 < CHUNKS)
            def _stage_ahead():
                stage(c + 2)

        return carry

    lax.fori_loop(0, CHUNKS + 2, it, 0)

    for k in range(CHUNKS - NBUF, CHUNKS):
        wait_store(k % NBUF)


@jax.jit
def _embed(i0, i1, i2, xb, yb, tlo, thi):
    mesh = plsc.VectorSubcoreMesh(core_axis_name="c", subcore_axis_name="s",
                                  num_cores=NC, num_subcores=NS)
    f = pl.kernel(
        _embed_body,
        out_type=jax.ShapeDtypeStruct((N, HIDDEN), jnp.float32),
        mesh=mesh,
        compiler_params=pltpu.CompilerParams(use_tc_tiling_on_sc=False,
                                             needs_layout_passes=False),
        scratch_types=[
            pltpu.VMEM((NBUF, T), jnp.int32),
            pltpu.VMEM((NBUF, T), jnp.int32),
            pltpu.VMEM((NBUF, T), jnp.int32),
            pltpu.VMEM((NBUF, T, HALF), jnp.bfloat16),
            pltpu.VMEM((NBUF, T, HALF), jnp.bfloat16),
            pltpu.VMEM((NBUF, T, HIDDEN), jnp.float32),
            pltpu.SemaphoreType.DMA((NBUF,)),
            pltpu.SemaphoreType.DMA((NBUF,)),
            pltpu.SemaphoreType.DMA((NBUF,)),
            pltpu.SemaphoreType.DMA((NBUF,)),
        ],
    )
    return f(i0, i1, i2, xb, yb, tlo, thi)


def _permcols(tbl):
    v, w = tbl.shape
    nb = w // 32
    t4 = tbl.reshape(v, nb, 2, L).transpose(0, 1, 3, 2)
    return t4.reshape(v, w).astype(jnp.bfloat16)


def kernel(input_states, x_embedding, y_embedding, type_embedding):
    inp = input_states.reshape(N, 3).astype(jnp.int32)
    i0 = inp[:, 0]
    i1 = inp[:, 1]
    i2 = inp[:, 2]
    xb = _permcols(x_embedding)
    yb = _permcols(y_embedding)
    tlo = _permcols(type_embedding[:, :HALF])
    thi = _permcols(type_embedding[:, HALF:])
    out = _embed(i0, i1, i2, xb, yb, tlo, thi)
    return out.reshape(BATCH, SEQ, HIDDEN)


# speedup vs baseline: 2.0330x; 1.0026x over previous
import functools

import jax
import jax.numpy as jnp
from jax import lax
from jax.experimental import pallas as pl
from jax.experimental.pallas import tpu as pltpu
from jax.experimental.pallas import tpu_sc as plsc

BATCH, SEQ = 4096, 200
HIDDEN = 128
HALF = HIDDEN // 2
N = BATCH * SEQ
NC, NS = 2, 16
NW = NC * NS
PER_W = N // NW
T = 128
CHUNKS = PER_W // T
L = 16
NBUF = 4


def _embed_body(i0_hbm, i1_hbm, i2_hbm, x_hbm, y_hbm, tlo_hbm, thi_hbm,
                out_hbm, idx0, idx1, idx2, lobuf, hibuf, obuf,
                ssem, tsem, asem, osem):
    wid = lax.axis_index("s") * NC + lax.axis_index("c")
    base = wid * PER_W

    def stage(c):
        p = c % NBUF
        src = pl.ds(base + c * T, T)
        pltpu.async_copy(i0_hbm.at[src], idx0.at[p], ssem.at[p])
        pltpu.async_copy(i1_hbm.at[src], idx1.at[p], ssem.at[p])
        pltpu.async_copy(i2_hbm.at[src], idx2.at[p], ssem.at[p])

    def wait_stage(p):
        dummy = pl.ds(0, T)
        pltpu.make_async_copy(i0_hbm.at[dummy], idx0.at[p], ssem.at[p]).wait()
        pltpu.make_async_copy(i1_hbm.at[dummy], idx1.at[p], ssem.at[p]).wait()
        pltpu.make_async_copy(i2_hbm.at[dummy], idx2.at[p], ssem.at[p]).wait()

    def bump(p):
        def bbody(i, carry):
            s = pl.ds(i * L, L)
            idx0[p, s] = idx0[p, s] + 1
            idx1[p, s] = idx1[p, s] + 1
            idx2[p, s] = idx2[p, s] + 1
            return carry
        lax.fori_loop(0, T // L, bbody, 0, unroll=True)

    def fire_writes(c):
        p = c % NBUF
        pltpu.async_copy(x_hbm.at[idx0.at[p]], lobuf.at[p], tsem.at[p])
        pltpu.async_copy(y_hbm.at[idx1.at[p]], hibuf.at[p], tsem.at[p])

    def wait_writes(p):
        pltpu.make_async_copy(x_hbm.at[idx0.at[p]], lobuf.at[p],
                              tsem.at[p]).wait()
        pltpu.make_async_copy(y_hbm.at[idx1.at[p]], hibuf.at[p],
                              tsem.at[p]).wait()

    def fire_adds(c):
        p = c % NBUF
        pltpu.async_copy(tlo_hbm.at[idx2.at[p]], lobuf.at[p], asem.at[p],
                         add=True)
        pltpu.async_copy(thi_hbm.at[idx2.at[p]], hibuf.at[p], asem.at[p],
                         add=True)

    def wait_adds(p):
        pltpu.make_async_copy(tlo_hbm.at[idx2.at[p]], lobuf.at[p],
                              asem.at[p]).wait()
        pltpu.make_async_copy(thi_hbm.at[idx2.at[p]], hibuf.at[p],
                              asem.at[p]).wait()

    def combine(r):
        lo = lobuf.at[r]
        hi = hibuf.at[r]
        ob = obuf.at[r]

        def vbody(t, carry):
            for j in range(2):
                ab = lo[t, pl.ds(32 * j, 32)]
                a, b = plsc.unpack(ab, format=plsc.PackFormat.INTERLEAVED)
                ob[t, pl.ds(32 * j, L)] = a
                ob[t, pl.ds(32 * j + L, L)] = b
                cd = hi[t, pl.ds(32 * j, 32)]
                cc, dd = plsc.unpack(cd, format=plsc.PackFormat.INTERLEAVED)
                ob[t, pl.ds(HALF + 32 * j, L)] = cc
                ob[t, pl.ds(HALF + 32 * j + L, L)] = dd
            return carry

        lax.fori_loop(0, T, vbody, 0, unroll=8)

    def fire_store(c):
        p = c % NBUF
        pltpu.async_copy(obuf.at[p], out_hbm.at[pl.ds(base + c * T, T)],
                         osem.at[p])

    def wait_store(p):
        pltpu.make_async_copy(obuf.at[p], out_hbm.at[pl.ds(base, T)],
                              osem.at[p]).wait()

    stage(0)
    stage(1)

    def it(c, carry):
        @pl.when(jnp.logical_and(c >= 1, c <= CHUNKS))
        def _adds():
            q = (c - 1) % NBUF
            wait_writes(q)
            fire_adds(c - 1)

        @pl.when(c >= 2)
        def _back():
            r = (c - 2) % NBUF
            wait_adds(r)
            combine(r)
            fire_store(c - 2)

        @pl.when(c < CHUNKS)
        def _front():
            p = c % NBUF
            wait_stage(p)
            bump(p)

            @pl.when(c >= NBUF)
            def _reuse():
                wait_store(p)

            fire_writes(c)

            @pl.when(c + 2 < CHUNKS)
            def _stage_ahead():
                stage(c + 2)

        return carry

    lax.fori_loop(0, CHUNKS + 2, it, 0)

    for k in range(CHUNKS - NBUF, CHUNKS):
        wait_store(k % NBUF)


@jax.jit
def _embed(i0, i1, i2, xb, yb, tlo, thi):
    mesh = plsc.VectorSubcoreMesh(core_axis_name="c", subcore_axis_name="s",
                                  num_cores=NC, num_subcores=NS)
    f = pl.kernel(
        _embed_body,
        out_type=jax.ShapeDtypeStruct((N, HIDDEN), jnp.float32),
        mesh=mesh,
        compiler_params=pltpu.CompilerParams(use_tc_tiling_on_sc=False,
                                             needs_layout_passes=False),
        scratch_types=[
            pltpu.VMEM((NBUF, T), jnp.int32),
            pltpu.VMEM((NBUF, T), jnp.int32),
            pltpu.VMEM((NBUF, T), jnp.int32),
            pltpu.VMEM((NBUF, T, HALF), jnp.bfloat16),
            pltpu.VMEM((NBUF, T, HALF), jnp.bfloat16),
            pltpu.VMEM((NBUF, T, HIDDEN), jnp.float32),
            pltpu.SemaphoreType.DMA((NBUF,)),
            pltpu.SemaphoreType.DMA((NBUF,)),
            pltpu.SemaphoreType.DMA((NBUF,)),
            pltpu.SemaphoreType.DMA((NBUF,)),
        ],
    )
    return f(i0, i1, i2, xb, yb, tlo, thi)


def _permcols(tbl):
    v, w = tbl.shape
    nb = w // 32
    t4 = tbl.reshape(v, nb, 2, L).transpose(0, 1, 3, 2)
    return t4.reshape(v, w).astype(jnp.bfloat16)


def kernel(input_states, x_embedding, y_embedding, type_embedding):
    inp = input_states.reshape(N, 3).astype(jnp.int32)
    i0 = inp[:, 0]
    i1 = inp[:, 1]
    i2 = inp[:, 2]
    xb = _permcols(x_embedding)
    yb = _permcols(y_embedding)
    tlo = _permcols(type_embedding[:, :HALF])
    thi = _permcols(type_embedding[:, HALF:])
    out = _embed(i0, i1, i2, xb, yb, tlo, thi)
    return out.reshape(BATCH, SEQ, HIDDEN)


# speedup vs baseline: 2.5174x; 1.2382x over previous
import functools

import jax
import jax.numpy as jnp
from jax import lax
from jax.experimental import pallas as pl
from jax.experimental.pallas import tpu as pltpu
from jax.experimental.pallas import tpu_sc as plsc

BATCH, SEQ = 4096, 200
HIDDEN = 128
HALF = HIDDEN // 2
N = BATCH * SEQ
NC, NS = 2, 16
NW = NC * NS
PER_W = N // NW
T = 128
CHUNKS = PER_W // T
L = 16
NIDX = 4
NB2 = 2


def _embed_body(i0_hbm, i1_hbm, i2_hbm,
                xf_hbm, yf_hbm, tlof_hbm, thif_hbm,
                xb_hbm, yb_hbm, tlob_hbm, thib_hbm,
                out_hbm,
                idx0, idx1, idx2,
                flobuf, fhibuf, blobuf, bhibuf, obuf,
                ssem, ftsem, fasem, fosem, btsem, basem, bosem):
    wid = lax.axis_index("s") * NC + lax.axis_index("c")
    base = wid * PER_W

    def stage(c):
        p = c % NIDX
        src = pl.ds(base + c * T, T)
        pltpu.async_copy(i0_hbm.at[src], idx0.at[p], ssem.at[p])
        pltpu.async_copy(i1_hbm.at[src], idx1.at[p], ssem.at[p])
        pltpu.async_copy(i2_hbm.at[src], idx2.at[p], ssem.at[p])

    def wait_stage(p):
        dummy = pl.ds(0, T)
        pltpu.make_async_copy(i0_hbm.at[dummy], idx0.at[p], ssem.at[p]).wait()
        pltpu.make_async_copy(i1_hbm.at[dummy], idx1.at[p], ssem.at[p]).wait()
        pltpu.make_async_copy(i2_hbm.at[dummy], idx2.at[p], ssem.at[p]).wait()

    def bump(p):
        def bbody(i, carry):
            s = pl.ds(i * L, L)
            idx0[p, s] = idx0[p, s] + 1
            idx1[p, s] = idx1[p, s] + 1
            idx2[p, s] = idx2[p, s] + 1
            return carry
        lax.fori_loop(0, T // L, bbody, 0, unroll=True)

    def slot2(c):
        return (c // 2) % NB2

    def f_fire_writes(c):
        p, q = slot2(c), c % NIDX
        pltpu.async_copy(xf_hbm.at[idx0.at[q]], flobuf.at[p], ftsem.at[p])
        pltpu.async_copy(yf_hbm.at[idx1.at[q]], fhibuf.at[p], ftsem.at[p])

    def f_wait_writes(c):
        p, q = slot2(c), c % NIDX
        pltpu.make_async_copy(xf_hbm.at[idx0.at[q]], flobuf.at[p],
                              ftsem.at[p]).wait()
        pltpu.make_async_copy(yf_hbm.at[idx1.at[q]], fhibuf.at[p],
                              ftsem.at[p]).wait()

    def f_fire_adds(c):
        p, q = slot2(c), c % NIDX
        pltpu.async_copy(tlof_hbm.at[idx2.at[q]], flobuf.at[p], fasem.at[p],
                         add=True)
        pltpu.async_copy(thif_hbm.at[idx2.at[q]], fhibuf.at[p], fasem.at[p],
                         add=True)

    def f_wait_adds(c):
        p, q = slot2(c), c % NIDX
        pltpu.make_async_copy(tlof_hbm.at[idx2.at[q]], flobuf.at[p],
                              fasem.at[p]).wait()
        pltpu.make_async_copy(thif_hbm.at[idx2.at[q]], fhibuf.at[p],
                              fasem.at[p]).wait()

    def f_fire_store(c):
        p = slot2(c)
        rows = pl.ds(base + c * T, T)
        pltpu.async_copy(flobuf.at[p], out_hbm.at[rows, pl.ds(0, HALF)],
                         fosem.at[p])
        pltpu.async_copy(fhibuf.at[p], out_hbm.at[rows, pl.ds(HALF, HALF)],
                         fosem.at[p])

    def f_wait_store(p):
        rows = pl.ds(base, T)
        pltpu.make_async_copy(flobuf.at[p], out_hbm.at[rows, pl.ds(0, HALF)],
                              fosem.at[p]).wait()
        pltpu.make_async_copy(fhibuf.at[p],
                              out_hbm.at[rows, pl.ds(HALF, HALF)],
                              fosem.at[p]).wait()

    def b_fire_writes(c):
        p, q = slot2(c), c % NIDX
        pltpu.async_copy(xb_hbm.at[idx0.at[q]], blobuf.at[p], btsem.at[p])
        pltpu.async_copy(yb_hbm.at[idx1.at[q]], bhibuf.at[p], btsem.at[p])

    def b_wait_writes(c):
        p, q = slot2(c), c % NIDX
        pltpu.make_async_copy(xb_hbm.at[idx0.at[q]], blobuf.at[p],
                              btsem.at[p]).wait()
        pltpu.make_async_copy(yb_hbm.at[idx1.at[q]], bhibuf.at[p],
                              btsem.at[p]).wait()

    def b_fire_adds(c):
        p, q = slot2(c), c % NIDX
        pltpu.async_copy(tlob_hbm.at[idx2.at[q]], blobuf.at[p], basem.at[p],
                         add=True)
        pltpu.async_copy(thib_hbm.at[idx2.at[q]], bhibuf.at[p], basem.at[p],
                         add=True)

    def b_wait_adds(c):
        p, q = slot2(c), c % NIDX
        pltpu.make_async_copy(tlob_hbm.at[idx2.at[q]], blobuf.at[p],
                              basem.at[p]).wait()
        pltpu.make_async_copy(thib_hbm.at[idx2.at[q]], bhibuf.at[p],
                              basem.at[p]).wait()

    def combine(r):
        lo = blobuf.at[r]
        hi = bhibuf.at[r]
        ob = obuf.at[r]

        def vbody(t, carry):
            for j in range(2):
                ab = lo[t, pl.ds(32 * j, 32)]
                a, b = plsc.unpack(ab, format=plsc.PackFormat.INTERLEAVED)
                ob[t, pl.ds(32 * j, L)] = a
                ob[t, pl.ds(32 * j + L, L)] = b
                cd = hi[t, pl.ds(32 * j, 32)]
                cc, dd = plsc.unpack(cd, format=plsc.PackFormat.INTERLEAVED)
                ob[t, pl.ds(HALF + 32 * j, L)] = cc
                ob[t, pl.ds(HALF + 32 * j + L, L)] = dd
            return carry

        lax.fori_loop(0, T, vbody, 0, unroll=8)

    def b_fire_store(c):
        p = slot2(c)
        pltpu.async_copy(obuf.at[p], out_hbm.at[pl.ds(base + c * T, T)],
                         bosem.at[p])

    def b_wait_store(p):
        pltpu.make_async_copy(obuf.at[p], out_hbm.at[pl.ds(base, T)],
                              bosem.at[p]).wait()

    stage(0)
    stage(1)

    def it(c, carry):
        even = (c % 2) == 0

        @pl.when(jnp.logical_and(c >= 1, c <= CHUNKS))
        def _adds():
            d = c - 1

            @pl.when(even)
            def _f():
                f_wait_writes(d)
                f_fire_adds(d)

            @pl.when(jnp.logical_not(even))
            def _b():
                b_wait_writes(d)
                b_fire_adds(d)

        @pl.when(c >= 2)
        def _back():
            d = c - 2

            @pl.when(even)
            def _b():
                b_wait_adds(d)
                combine(slot2(d))
                b_fire_store(d)

            @pl.when(jnp.logical_not(even))
            def _f():
                f_wait_adds(d)
                f_fire_store(d)

        @pl.when(c < CHUNKS)
        def _front():
            wait_stage(c % NIDX)
            bump(c % NIDX)

            @pl.when(even)
            def _b():
                @pl.when(c >= 2 * NB2)
                def _reuse():
                    b_wait_store(slot2(c))
                b_fire_writes(c)

            @pl.when(jnp.logical_not(even))
            def _f():
                @pl.when(c >= 2 * NB2)
                def _reuse():
                    f_wait_store(slot2(c))
                f_fire_writes(c)

            @pl.when(c + 2 < CHUNKS)
            def _stage_ahead():
                stage(c + 2)

        return carry

    lax.fori_loop(0, CHUNKS + 2, it, 0)

    for k in range(CHUNKS - 2 * NB2, CHUNKS):
        if k % 2 == 0:
            b_wait_store(slot2(k))
        else:
            f_wait_store(slot2(k))


@jax.jit
def _embed(i0, i1, i2, xf, yf, tlof, thif, xb, yb, tlob, thib):
    mesh = plsc.VectorSubcoreMesh(core_axis_name="c", subcore_axis_name="s",
                                  num_cores=NC, num_subcores=NS)
    f = pl.kernel(
        _embed_body,
        out_type=jax.ShapeDtypeStruct((N, HIDDEN), jnp.float32),
        mesh=mesh,
        compiler_params=pltpu.CompilerParams(use_tc_tiling_on_sc=False,
                                             needs_layout_passes=False),
        scratch_types=[
            pltpu.VMEM((NIDX, T), jnp.int32),
            pltpu.VMEM((NIDX, T), jnp.int32),
            pltpu.VMEM((NIDX, T), jnp.int32),
            pltpu.VMEM((NB2, T, HALF), jnp.float32),
            pltpu.VMEM((NB2, T, HALF), jnp.float32),
            pltpu.VMEM((NB2, T, HALF), jnp.bfloat16),
            pltpu.VMEM((NB2, T, HALF), jnp.bfloat16),
            pltpu.VMEM((NB2, T, HIDDEN), jnp.float32),
            pltpu.SemaphoreType.DMA((NIDX,)),
            pltpu.SemaphoreType.DMA((NB2,)),
            pltpu.SemaphoreType.DMA((NB2,)),
            pltpu.SemaphoreType.DMA((NB2,)),
            pltpu.SemaphoreType.DMA((NB2,)),
            pltpu.SemaphoreType.DMA((NB2,)),
            pltpu.SemaphoreType.DMA((NB2,)),
        ],
    )
    return f(i0, i1, i2, xf, yf, tlof, thif, xb, yb, tlob, thib)


def _permcols(tbl):
    v, w = tbl.shape
    nb = w // 32
    t4 = tbl.reshape(v, nb, 2, L).transpose(0, 1, 3, 2)
    return t4.reshape(v, w).astype(jnp.bfloat16)


def kernel(input_states, x_embedding, y_embedding, type_embedding):
    inp = input_states.reshape(N, 3).astype(jnp.int32)
    i0 = inp[:, 0]
    i1 = inp[:, 1]
    i2 = inp[:, 2]
    tlof = type_embedding[:, :HALF]
    thif = type_embedding[:, HALF:]
    xb = _permcols(x_embedding)
    yb = _permcols(y_embedding)
    tlob = _permcols(tlof)
    thib = _permcols(thif)
    out = _embed(i0, i1, i2, x_embedding, y_embedding, tlof, thif,
                 xb, yb, tlob, thib)
    return out.reshape(BATCH, SEQ, HIDDEN)
